# baseline (speedup 1.0000x reference)
import functools

import jax
import jax.numpy as jnp
from jax import lax
from jax.experimental import pallas as pl
from jax.experimental.pallas import tpu as pltpu
from jax.experimental.pallas import tpu_sc as plsc

N = 10000
E = 320000
D = 128
BLK = 1000
GRID = N // BLK

NC = 2
NS = 16
NW = NC * NS
R = 313
CH = 4000
RB = 128


def _stage1_body(x_ref, ph_ref, w0_ref, w1_ref, b_ref, ma_ref, mb_ref,
                 z_ref, a_ref, bb_ref):
    z = jnp.maximum(ph_ref[...] @ w1_ref[...] + x_ref[...] * w0_ref[...]
                    + b_ref[...], 0.0)
    z_ref[...] = z
    a_ref[...] = z @ ma_ref[...]
    bb_ref[...] = z @ mb_ref[...]


def _stage1(x, pre_h, enc_w0, enc_w1, enc_b, ma, mb):
    return pl.pallas_call(
        _stage1_body,
        grid=(GRID,),
        in_specs=[
            pl.BlockSpec((BLK, 1), lambda i: (i, 0)),
            pl.BlockSpec((BLK, D), lambda i: (i, 0)),
            pl.BlockSpec((1, D), lambda i: (0, 0)),
            pl.BlockSpec((D, D), lambda i: (0, 0)),
            pl.BlockSpec((1, D), lambda i: (0, 0)),
            pl.BlockSpec((D, D), lambda i: (0, 0)),
            pl.BlockSpec((D, D), lambda i: (0, 0)),
        ],
        out_specs=[
            pl.BlockSpec((BLK, D), lambda i: (i, 0)),
            pl.BlockSpec((BLK, D), lambda i: (i, 0)),
            pl.BlockSpec((BLK, D), lambda i: (i, 0)),
        ],
        out_shape=[
            jax.ShapeDtypeStruct((N, D), jnp.float32),
            jax.ShapeDtypeStruct((N, D), jnp.float32),
            jax.ShapeDtypeStruct((N, D), jnp.float32),
        ],
    )(x, pre_h, enc_w0, enc_w1, enc_b, ma, mb)


def _stage3_body(z_ref, a_ref, g_ref, mb_ref, u1_ref, u2_ref, ub_ref,
                 d1_ref, d2_ref, db_ref, h_ref, y_ref, hs_ref):
    g = g_ref[...]
    aggr = jnp.where(jnp.isneginf(g), 0.0,
                     jnp.maximum(a_ref[...] + g + mb_ref[...], 0.0))
    h = jnp.maximum(z_ref[...] @ u1_ref[...] + aggr @ u2_ref[...]
                    + ub_ref[...], 0.0)
    h_ref[...] = h
    y_ref[...] = z_ref[...] @ d1_ref[...] + h @ d2_ref[...] + db_ref[...]
    hs_ref[...] = jnp.sum(h, axis=0, keepdims=True)[None]


def _stage3(z, a, g, m_b, u1, u2, u_b, d1, d2, d_b):
    return pl.pallas_call(
        _stage3_body,
        grid=(GRID,),
        in_specs=[
            pl.BlockSpec((BLK, D), lambda i: (i, 0)),
            pl.BlockSpec((BLK, D), lambda i: (i, 0)),
            pl.BlockSpec((BLK, D), lambda i: (i, 0)),
            pl.BlockSpec((1, D), lambda i: (0, 0)),
            pl.BlockSpec((D, D), lambda i: (0, 0)),
            pl.BlockSpec((D, D), lambda i: (0, 0)),
            pl.BlockSpec((1, D), lambda i: (0, 0)),
            pl.BlockSpec((D, 1), lambda i: (0, 0)),
            pl.BlockSpec((D, 1), lambda i: (0, 0)),
            pl.BlockSpec((1, 1), lambda i: (0, 0)),
        ],
        out_specs=[
            pl.BlockSpec((BLK, D), lambda i: (i, 0)),
            pl.BlockSpec((BLK, 1), lambda i: (i, 0)),
            pl.BlockSpec((1, 1, D), lambda i: (i, 0, 0)),
        ],
        out_shape=[
            jax.ShapeDtypeStruct((N, D), jnp.float32),
            jax.ShapeDtypeStruct((N, 1), jnp.float32),
            jax.ShapeDtypeStruct((GRID, 1, D), jnp.float32),
        ],
    )(z, a, g, m_b, u1, u2, u_b, d1, d2, d_b)


def _edge_sc_body(src_hbm, dst_hbm, ea_hbm, b_hbm, w_hbm, g_hbm,
                  srcb, dstb, eab, flt_src, flt_dl, flt_ea, rows, wv,
                  g_loc, sem):
    wid = lax.axis_index("s") * NC + lax.axis_index("c")
    base = wid * R

    def _init(i, _):
        g_loc[pl.ds(i * 16, 16)] = jnp.full((16,), -jnp.inf, jnp.float32)
        return 0
    lax.fori_loop(0, R * D // 16, _init, 0)

    def _zero(i, _):
        flt_src[pl.ds(i * 16, 16)] = jnp.zeros((16,), jnp.int32)
        return 0
    lax.fori_loop(0, (CH + 16) // 16, _zero, 0)

    pltpu.sync_copy(w_hbm, wv)
    wregs = [wv[pl.ds(i * 16, 16)] for i in range(D // 16)]

    def _chunk(c, _):
        off = c * CH
        pltpu.sync_copy(src_hbm.at[pl.ds(off, CH)], srcb)
        pltpu.sync_copy(dst_hbm.at[pl.ds(off, CH)], dstb)
        pltpu.sync_copy(ea_hbm.at[pl.ds(off, CH)], eab)

        def _filt(j, cnt):
            lv = dstb[pl.ds(j * 16, 16)] - base
            m = (lv >= 0) & (lv < R)
            cum = plsc.cumsum(m.astype(jnp.int32))
            pos = cnt + cum - 1
            plsc.store_scatter(flt_dl, [pos], lv, mask=m)
            plsc.store_scatter(flt_src, [pos], srcb[pl.ds(j * 16, 16)],
                               mask=m)
            plsc.store_scatter(flt_ea, [pos], eab[pl.ds(j * 16, 16)],
                               mask=m)
            return cnt + plsc.all_reduce_population_count(m)[0]
        cnt = lax.fori_loop(0, CH // 16, _filt, jnp.int32(0))

        def _sub(k, _):
            idx = flt_src.at[pl.ds(k * RB, RB)]
            pltpu.async_copy(b_hbm.at[idx], rows, sem).wait()
            nedge = jnp.minimum(cnt - k * RB, RB)

            def _edge(i, _):
                p = k * RB + i
                dl = flt_dl[pl.ds(p, 16)][0]
                eai = flt_ea[pl.ds(p, 16)][0]
                gb = dl * D
                for cs in range(D // 16):
                    o = cs * 16
                    cand = rows[i, pl.ds(o, 16)] + eai * wregs[cs]
                    g_loc[pl.ds(gb + o, 16)] = jnp.maximum(
                        g_loc[pl.ds(gb + o, 16)], cand)
                return 0
            lax.fori_loop(0, nedge, _edge, 0)
            return 0
        lax.fori_loop(0, (cnt + RB - 1) // RB, _sub, 0)
        return 0
    lax.fori_loop(0, E // CH, _chunk, 0)

    pltpu.sync_copy(g_loc, g_hbm.at[pl.ds(base * D, R * D)])


_edge_sc = functools.partial(
    pl.kernel,
    out_type=jax.ShapeDtypeStruct((NW * R * D,), jnp.float32),
    mesh=plsc.VectorSubcoreMesh(core_axis_name="c", subcore_axis_name="s"),
    compiler_params=pltpu.CompilerParams(needs_layout_passes=False),
    scratch_types=[
        pltpu.VMEM((CH,), jnp.int32),
        pltpu.VMEM((CH,), jnp.int32),
        pltpu.VMEM((CH,), jnp.float32),
        pltpu.VMEM((CH + 16,), jnp.int32),
        pltpu.VMEM((CH + 16,), jnp.int32),
        pltpu.VMEM((CH + 16,), jnp.float32),
        pltpu.VMEM((RB, D), jnp.float32),
        pltpu.VMEM((D,), jnp.float32),
        pltpu.VMEM((R * D,), jnp.float32),
        pltpu.SemaphoreType.DMA,
    ],
)(_edge_sc_body)


def _edge_stage(b_mat, src, dst, ea, w):
    g_flat = _edge_sc(src, dst, ea, b_mat, w)
    return g_flat.reshape(NW * R, D)[:N]


def kernel(x, pre_h, edge_index, edge_attr, enc_W, enc_b, M_W, M_b,
           U_W, U_b, dec_W, dec_b, term_W, term_b):
    enc_w0 = enc_W[0:1]
    enc_w1 = enc_W[1:]
    ma = M_W[0:D]
    mb = M_W[D:2 * D]
    w = M_W[2 * D]

    z, a, b_mat = _stage1(x, pre_h, enc_w0, enc_w1, enc_b.reshape(1, D),
                          ma, mb)

    g = _edge_stage(b_mat, edge_index[0], edge_index[1], edge_attr[:, 0], w)

    h, y, hs = _stage3(z, a, g, M_b.reshape(1, D), U_W[0:D], U_W[D:],
                       U_b.reshape(1, D), dec_W[0:D], dec_W[D:],
                       dec_b.reshape(1, 1))

    h_mean = jnp.sum(hs[:, 0, :], axis=0, keepdims=True) / N
    tau = h_mean @ (term_W[0:D] + term_W[D:]) + term_b
    return (h, y, tau)


# speedup vs baseline: 1.9767x; 1.9767x over previous
import functools

import jax
import jax.numpy as jnp
from jax import lax
from jax.experimental import pallas as pl
from jax.experimental.pallas import tpu as pltpu
from jax.experimental.pallas import tpu_sc as plsc

N = 10000
E = 320000
D = 128
BLK = 1000
GRID = N // BLK

NC = 2
NS = 16
NW = NC * NS
R = 313
CH = 8000
RB = 128
FLT = ((CH + RB - 1) // RB + 1) * RB


def _stage1_body(x_ref, ph_ref, ew_ref, b_ref, ma_ref, mb_ref,
                 z_ref, a_ref, bb_ref):
    zin = jnp.concatenate([x_ref[...], ph_ref[...]], axis=-1)
    z = jnp.maximum(zin @ ew_ref[...] + b_ref[...], 0.0)
    z_ref[...] = z
    a_ref[...] = z @ ma_ref[...]
    bb_ref[...] = z @ mb_ref[...]


def _stage1(x, pre_h, enc_W, enc_b, ma, mb):
    return pl.pallas_call(
        _stage1_body,
        grid=(GRID,),
        in_specs=[
            pl.BlockSpec((BLK, 1), lambda i: (i, 0)),
            pl.BlockSpec((BLK, D), lambda i: (i, 0)),
            pl.BlockSpec((D + 1, D), lambda i: (0, 0)),
            pl.BlockSpec((1, D), lambda i: (0, 0)),
            pl.BlockSpec((D, D), lambda i: (0, 0)),
            pl.BlockSpec((D, D), lambda i: (0, 0)),
        ],
        out_specs=[
            pl.BlockSpec((BLK, D), lambda i: (i, 0)),
            pl.BlockSpec((BLK, D), lambda i: (i, 0)),
            pl.BlockSpec((BLK, D), lambda i: (i, 0)),
        ],
        out_shape=[
            jax.ShapeDtypeStruct((N, D), jnp.float32),
            jax.ShapeDtypeStruct((N, D), jnp.float32),
            jax.ShapeDtypeStruct((N, D), jnp.float32),
        ],
    )(x, pre_h, enc_W, enc_b, ma, mb)


def _stage3_body(z_ref, a_ref, g_ref, mb_ref, uw_ref, ub_ref,
                 dw_ref, db_ref, t1_ref, h_ref, y_ref, hs_ref, ts_ref):
    g = g_ref[...]
    aggr = jnp.where(jnp.isneginf(g), 0.0,
                     jnp.maximum(a_ref[...] + g + mb_ref[...], 0.0))
    za = jnp.concatenate([z_ref[...], aggr], axis=-1)
    h = jnp.maximum(za @ uw_ref[...] + ub_ref[...], 0.0)
    h_ref[...] = h
    zh = jnp.concatenate([z_ref[...], h], axis=-1)
    y_ref[...] = zh @ dw_ref[...] + db_ref[...]
    hs_ref[...] = jnp.sum(h, axis=0, keepdims=True)[None]
    ts_ref[...] = jnp.sum(h @ t1_ref[...], axis=0, keepdims=True)[None]


def _stage3(z, a, g, m_b, u_w, u_b, d_w, d_b, t1):
    return pl.pallas_call(
        _stage3_body,
        grid=(GRID,),
        in_specs=[
            pl.BlockSpec((BLK, D), lambda i: (i, 0)),
            pl.BlockSpec((BLK, D), lambda i: (i, 0)),
            pl.BlockSpec((BLK, D), lambda i: (i, 0)),
            pl.BlockSpec((1, D), lambda i: (0, 0)),
            pl.BlockSpec((2 * D, D), lambda i: (0, 0)),
            pl.BlockSpec((1, D), lambda i: (0, 0)),
            pl.BlockSpec((2 * D, 1), lambda i: (0, 0)),
            pl.BlockSpec((1, 1), lambda i: (0, 0)),
            pl.BlockSpec((D, 1), lambda i: (0, 0)),
        ],
        out_specs=[
            pl.BlockSpec((BLK, D), lambda i: (i, 0)),
            pl.BlockSpec((BLK, 1), lambda i: (i, 0)),
            pl.BlockSpec((1, 1, D), lambda i: (i, 0, 0)),
            pl.BlockSpec((1, 1, 1), lambda i: (i, 0, 0)),
        ],
        out_shape=[
            jax.ShapeDtypeStruct((N, D), jnp.float32),
            jax.ShapeDtypeStruct((N, 1), jnp.float32),
            jax.ShapeDtypeStruct((GRID, 1, D), jnp.float32),
            jax.ShapeDtypeStruct((GRID, 1, 1), jnp.float32),
        ],
    )(z, a, g, m_b, u_w, u_b, d_w, d_b, t1)


def _edge_sc_body(src_hbm, dst_hbm, ea_hbm, b_hbm, w_hbm, g_hbm,
                  srcb, dstb, eab, flt_src, flt_dl, flt_ea, rows, wv,
                  g_loc, sem):
    wid = lax.axis_index("s") * NC + lax.axis_index("c")
    base = wid * R

    def _init(i, _):
        g_loc[pl.ds(i * 16, 16)] = jnp.full((16,), -jnp.inf, jnp.float32)
        return 0
    lax.fori_loop(0, (R + 1) * D // 16, _init, 0)

    def _zero(i, _):
        flt_src[pl.ds(i * 16, 16)] = jnp.zeros((16,), jnp.int32)
        flt_dl[pl.ds(i * 16, 16)] = jnp.full((16,), R, jnp.int32)
        flt_ea[pl.ds(i * 16, 16)] = jnp.zeros((16,), jnp.float32)
        return 0
    lax.fori_loop(0, FLT // 16, _zero, 0)

    pltpu.sync_copy(w_hbm, wv)
    wregs = [wv[pl.ds(i * 16, 16)] for i in range(D // 16)]

    def _chunk(c, _):
        off = c * CH
        pltpu.sync_copy(src_hbm.at[pl.ds(off, CH)], srcb)
        pltpu.sync_copy(dst_hbm.at[pl.ds(off, CH)], dstb)
        pltpu.sync_copy(ea_hbm.at[pl.ds(off, CH)], eab)

        def _filt(j, cnt):
            lv = dstb[pl.ds(j * 16, 16)] - base
            m = (lv >= 0) & (lv < R)
            pc = plsc.all_reduce_population_count(m)[0]

            @pl.when(pc > 0)
            def _():
                cum = plsc.cumsum(m.astype(jnp.int32))
                pos = cnt + cum - 1
                plsc.store_scatter(flt_dl, [pos], lv, mask=m)
                plsc.store_scatter(flt_src, [pos], srcb[pl.ds(j * 16, 16)],
                                   mask=m)
                plsc.store_scatter(flt_ea, [pos], eab[pl.ds(j * 16, 16)],
                                   mask=m)
            return cnt + pc
        cnt = lax.fori_loop(0, CH // 16, _filt, jnp.int32(0))

        def _sub(k, _):
            idx = flt_src.at[pl.ds(k * RB, RB)]
            pltpu.async_copy(b_hbm.at[idx], rows, sem).wait()
            ngrp = (jnp.minimum(cnt - k * RB, RB) + 15) // 16

            def _grp(jj, _):
                p0 = k * RB + jj * 16
                dlv = flt_dl[pl.ds(p0, 16)]
                eav = flt_ea[pl.ds(p0, 16)]
                for i in range(16):
                    gb = dlv[i] * D
                    ri = jj * 16 + i
                    eai = eav[i]
                    for cs in range(D // 16):
                        o = cs * 16
                        cand = rows[ri, pl.ds(o, 16)] + eai * wregs[cs]
                        g_loc[pl.ds(gb + o, 16)] = jnp.maximum(
                            g_loc[pl.ds(gb + o, 16)], cand)
                return 0
            lax.fori_loop(0, ngrp, _grp, 0)
            return 0
        lax.fori_loop(0, (cnt + RB - 1) // RB, _sub, 0)
        return 0
    lax.fori_loop(0, E // CH, _chunk, 0)

    pltpu.sync_copy(g_loc.at[pl.ds(0, R * D)], g_hbm.at[pl.ds(base * D, R * D)])


@functools.cache
def _edge_sc():
    return functools.partial(
        pl.kernel,
        out_type=jax.ShapeDtypeStruct((NW * R * D,), jnp.float32),
        mesh=plsc.VectorSubcoreMesh(core_axis_name="c", subcore_axis_name="s",
                                    num_cores=NC, num_subcores=NS),
        compiler_params=pltpu.CompilerParams(needs_layout_passes=False),
        scratch_types=[
        pltpu.VMEM((CH,), jnp.int32),
        pltpu.VMEM((CH,), jnp.int32),
        pltpu.VMEM((CH,), jnp.float32),
        pltpu.VMEM((FLT,), jnp.int32),
        pltpu.VMEM((FLT,), jnp.int32),
        pltpu.VMEM((FLT,), jnp.float32),
        pltpu.VMEM((RB, D), jnp.float32),
        pltpu.VMEM((D,), jnp.float32),
        pltpu.VMEM(((R + 1) * D,), jnp.float32),
            pltpu.SemaphoreType.DMA,
        ],
    )(_edge_sc_body)


def _edge_stage(b_mat, src, dst, ea, w):
    g_flat = _edge_sc()(src, dst, ea, b_mat, w)
    return g_flat.reshape(NW * R, D)[:N]


def kernel(x, pre_h, edge_index, edge_attr, enc_W, enc_b, M_W, M_b,
           U_W, U_b, dec_W, dec_b, term_W, term_b):
    ma = M_W[0:D]
    mb = M_W[D:2 * D]
    w = M_W[2 * D]

    z, a, b_mat = _stage1(x, pre_h, enc_W, enc_b.reshape(1, D), ma, mb)

    g = _edge_stage(b_mat, edge_index[0], edge_index[1], edge_attr[:, 0], w)

    h, y, hs, ts = _stage3(z, a, g, M_b.reshape(1, D), U_W,
                           U_b.reshape(1, D), dec_W, dec_b.reshape(1, 1),
                           term_W[0:D])

    h_mean = jnp.sum(hs[:, 0, :], axis=0, keepdims=True) / N
    tau = (jnp.sum(ts[:, 0, :], axis=0, keepdims=True) / N
           + h_mean @ term_W[D:] + term_b)
    return (h, y, tau)


# speedup vs baseline: 1.9777x; 1.0005x over previous
import functools

import jax
import jax.numpy as jnp
from jax import lax
from jax.experimental import pallas as pl
from jax.experimental.pallas import tpu as pltpu
from jax.experimental.pallas import tpu_sc as plsc

N = 10000
E = 320000
D = 128
BLK = 1000
GRID = N // BLK

NC = 2
NS = 16
NW = NC * NS
R = 313
CH = 8000
RB = 128
FLT = ((CH + RB - 1) // RB + 1) * RB


def _stage1_body(x_ref, ph_ref, ew_ref, b_ref, ma_ref, mb_ref,
                 z_ref, a_ref, bb_ref):
    zin = jnp.concatenate([x_ref[...], ph_ref[...]], axis=-1)
    z = jnp.maximum(zin @ ew_ref[...] + b_ref[...], 0.0)
    z_ref[...] = z
    a_ref[...] = z @ ma_ref[...]
    bb_ref[...] = z @ mb_ref[...]


def _stage1(x, pre_h, enc_W, enc_b, ma, mb):
    return pl.pallas_call(
        _stage1_body,
        grid=(GRID,),
        in_specs=[
            pl.BlockSpec((BLK, 1), lambda i: (i, 0)),
            pl.BlockSpec((BLK, D), lambda i: (i, 0)),
            pl.BlockSpec((D + 1, D), lambda i: (0, 0)),
            pl.BlockSpec((1, D), lambda i: (0, 0)),
            pl.BlockSpec((D, D), lambda i: (0, 0)),
            pl.BlockSpec((D, D), lambda i: (0, 0)),
        ],
        out_specs=[
            pl.BlockSpec((BLK, D), lambda i: (i, 0)),
            pl.BlockSpec((BLK, D), lambda i: (i, 0)),
            pl.BlockSpec((BLK, D), lambda i: (i, 0)),
        ],
        out_shape=[
            jax.ShapeDtypeStruct((N, D), jnp.float32),
            jax.ShapeDtypeStruct((N, D), jnp.float32),
            jax.ShapeDtypeStruct((N, D), jnp.float32),
        ],
    )(x, pre_h, enc_W, enc_b, ma, mb)


def _stage3_body(z_ref, a_ref, g_ref, mb_ref, uw_ref, ub_ref,
                 dw_ref, db_ref, t1_ref, h_ref, y_ref, hs_ref, ts_ref):
    g = g_ref[...]
    aggr = jnp.where(jnp.isneginf(g), 0.0,
                     jnp.maximum(a_ref[...] + g + mb_ref[...], 0.0))
    za = jnp.concatenate([z_ref[...], aggr], axis=-1)
    h = jnp.maximum(za @ uw_ref[...] + ub_ref[...], 0.0)
    h_ref[...] = h
    zh = jnp.concatenate([z_ref[...], h], axis=-1)
    y_ref[...] = zh @ dw_ref[...] + db_ref[...]
    hs_ref[...] = jnp.sum(h, axis=0, keepdims=True)[None]
    ts_ref[...] = jnp.sum(h @ t1_ref[...], axis=0, keepdims=True)[None]


def _stage3(z, a, g, m_b, u_w, u_b, d_w, d_b, t1):
    return pl.pallas_call(
        _stage3_body,
        grid=(GRID,),
        in_specs=[
            pl.BlockSpec((BLK, D), lambda i: (i, 0)),
            pl.BlockSpec((BLK, D), lambda i: (i, 0)),
            pl.BlockSpec((BLK, D), lambda i: (i, 0)),
            pl.BlockSpec((1, D), lambda i: (0, 0)),
            pl.BlockSpec((2 * D, D), lambda i: (0, 0)),
            pl.BlockSpec((1, D), lambda i: (0, 0)),
            pl.BlockSpec((2 * D, 1), lambda i: (0, 0)),
            pl.BlockSpec((1, 1), lambda i: (0, 0)),
            pl.BlockSpec((D, 1), lambda i: (0, 0)),
        ],
        out_specs=[
            pl.BlockSpec((BLK, D), lambda i: (i, 0)),
            pl.BlockSpec((BLK, 1), lambda i: (i, 0)),
            pl.BlockSpec((1, 1, D), lambda i: (i, 0, 0)),
            pl.BlockSpec((1, 1, 1), lambda i: (i, 0, 0)),
        ],
        out_shape=[
            jax.ShapeDtypeStruct((N, D), jnp.float32),
            jax.ShapeDtypeStruct((N, 1), jnp.float32),
            jax.ShapeDtypeStruct((GRID, 1, D), jnp.float32),
            jax.ShapeDtypeStruct((GRID, 1, 1), jnp.float32),
        ],
    )(z, a, g, m_b, u_w, u_b, d_w, d_b, t1)


def _edge_sc_body(src_hbm, dst_hbm, ea_hbm, b_hbm, w_hbm, g_hbm,
                  srcb, dstb, eab, flt_src, flt_dl, flt_ea, rows, wv,
                  g_loc, sem):
    wid = lax.axis_index("s") * NC + lax.axis_index("c")
    base = wid * R

    def _init(i, _):
        g_loc[pl.ds(i * 16, 16)] = jnp.full((16,), -jnp.inf, jnp.float32)
        return 0
    lax.fori_loop(0, (R + 1) * D // 16, _init, 0)

    def _zero(i, _):
        flt_src[pl.ds(i * 16, 16)] = jnp.zeros((16,), jnp.int32)
        flt_dl[pl.ds(i * 16, 16)] = jnp.full((16,), R, jnp.int32)
        flt_ea[pl.ds(i * 16, 16)] = jnp.zeros((16,), jnp.float32)
        return 0
    lax.fori_loop(0, FLT // 16, _zero, 0)

    pltpu.sync_copy(w_hbm, wv)
    wregs = [wv[pl.ds(i * 16, 16)] for i in range(D // 16)]

    def _chunk(c, _):
        off = c * CH
        pltpu.sync_copy(src_hbm.at[pl.ds(off, CH)], srcb)
        pltpu.sync_copy(dst_hbm.at[pl.ds(off, CH)], dstb)
        pltpu.sync_copy(ea_hbm.at[pl.ds(off, CH)], eab)

        def _filt(j, cnt):
            lv = dstb[pl.ds(j * 16, 16)] - base
            m = (lv >= 0) & (lv < R)
            pc = plsc.all_reduce_population_count(m)[0]

            @pl.when(pc > 0)
            def _():
                cum = plsc.cumsum(m.astype(jnp.int32))
                pos = cnt + cum - 1
                plsc.store_scatter(flt_dl, [pos], lv, mask=m)
                plsc.store_scatter(flt_src, [pos], srcb[pl.ds(j * 16, 16)],
                                   mask=m)
                plsc.store_scatter(flt_ea, [pos], eab[pl.ds(j * 16, 16)],
                                   mask=m)
            return cnt + pc
        cnt = lax.fori_loop(0, CH // 16, _filt, jnp.int32(0))

        def _sub(k, _):
            idx = flt_src.at[pl.ds(k * RB, RB)]
            pltpu.async_copy(b_hbm.at[idx], rows, sem).wait()
            ngrp = (jnp.minimum(cnt - k * RB, RB) + 15) // 16

            def _grp(jj, _):
                p0 = k * RB + jj * 16
                dlv = flt_dl[pl.ds(p0, 16)]
                eav = flt_ea[pl.ds(p0, 16)]
                for i in range(16):
                    gb = dlv[i] * D
                    ri = jj * 16 + i
                    eai = eav[i]
                    for cs in range(D // 16):
                        o = cs * 16
                        cand = rows[ri, pl.ds(o, 16)] + eai * wregs[cs]
                        g_loc[pl.ds(gb + o, 16)] = jnp.maximum(
                            g_loc[pl.ds(gb + o, 16)], cand)
                return 0
            lax.fori_loop(0, ngrp, _grp, 0)
            return 0
        lax.fori_loop(0, (cnt + RB - 1) // RB, _sub, 0)
        return 0
    lax.fori_loop(0, E // CH, _chunk, 0)

    pltpu.sync_copy(g_loc.at[pl.ds(0, R * D)], g_hbm.at[pl.ds(base * D, R * D)])


@functools.cache
def _edge_sc():
    return functools.partial(
        pl.kernel,
        out_type=jax.ShapeDtypeStruct((NW * R * D,), jnp.float32),
        mesh=plsc.VectorSubcoreMesh(core_axis_name="c", subcore_axis_name="s",
                                    num_cores=NC, num_subcores=NS),
        compiler_params=pltpu.CompilerParams(needs_layout_passes=False),
        scratch_types=[
        pltpu.VMEM((CH,), jnp.int32),
        pltpu.VMEM((CH,), jnp.int32),
        pltpu.VMEM((CH,), jnp.float32),
        pltpu.VMEM((FLT,), jnp.int32),
        pltpu.VMEM((FLT,), jnp.int32),
        pltpu.VMEM((FLT,), jnp.float32),
        pltpu.VMEM((RB, D), jnp.float32),
        pltpu.VMEM((D,), jnp.float32),
        pltpu.VMEM(((R + 1) * D,), jnp.float32),
            pltpu.SemaphoreType.DMA,
        ],
    )(_edge_sc_body)


def _edge_stage(b_mat, src, dst, ea, w):
    g_flat = _edge_sc()(src, dst, ea, b_mat, w)
    return g_flat.reshape(NW * R, D)[:N]


def kernel(x, pre_h, edge_index, edge_attr, enc_W, enc_b, M_W, M_b,
           U_W, U_b, dec_W, dec_b, term_W, term_b):
    ma = M_W[0:D]
    mb = M_W[D:2 * D]
    w = M_W[2 * D]

    z, a, b_mat = _stage1(x, pre_h, enc_W, enc_b.reshape(1, D), ma, mb)

    ea_bf = edge_attr[:, 0].astype(jnp.bfloat16).astype(jnp.float32)
    w_bf = w.astype(jnp.bfloat16).astype(jnp.float32)
    g = _edge_stage(b_mat, edge_index[0], edge_index[1], ea_bf, w_bf)

    h, y, hs, ts = _stage3(z, a, g, M_b.reshape(1, D), U_W,
                           U_b.reshape(1, D), dec_W, dec_b.reshape(1, 1),
                           term_W[0:D])

    h_mean = jnp.sum(hs[:, 0, :], axis=0, keepdims=True) / N
    tau = (jnp.sum(ts[:, 0, :], axis=0, keepdims=True) / N
           + h_mean @ term_W[D:] + term_b)
    return (h, y, tau)
